# Initial kernel scaffold; baseline (speedup 1.0000x reference)
import jax
import jax.numpy as jnp
from jax.experimental import pallas as pl


def kernel(x, edge_index, edge_attr, ew1_w, ew1_b, root1, bias1, ew2_w, ew2_b, root2, bias2, head_cls_w, head_cls_b, head_reg_w, head_reg_b):
    raise NotImplementedError("write your pallas kernel here")


# speedup vs baseline: 5.1161x; 5.1161x over previous
import functools

import jax
import jax.numpy as jnp
from jax import lax
from jax.experimental import pallas as pl
from jax.experimental.pallas import tpu as pltpu
from jax.experimental.pallas import tpu_sc as plsc

N = 10000
E = 160000
D = 25
H = 32
TW = 3 * H

NC = 2
NS = 16
NW = NC * NS
CHUNK = 128
NCHUNKS = E // CHUNK
CH_BASE = NCHUNKS // NW
CH_REM = NCHUNKS % NW
ZCH = 200
NZCH = N // ZCH
ZBASE = NZCH // NS
ZREM = NZCH % NS

_f32 = jnp.float32


def _edge_body(W, with_count, table_hbm, src_hbm, dst_hbm, ea_hbm, out_hbm,
               src_idx, dst_idx, ea_v, rows_v, msg_v, zbuf, acc, gsem):
    c = lax.axis_index("c")
    s = lax.axis_index("s")
    wid = s * NC + c

    zero16 = jnp.zeros((16,), _f32)

    def zrow(i, carry):
        for j in range(W // 16):
            zbuf[i, pl.ds(j * 16, 16)] = zero16
        return carry

    lax.fori_loop(0, ZCH, zrow, 0)
    nz = ZBASE + jnp.where(s < ZREM, 1, 0)

    def zchunk(k, carry):
        b = (s + k * NS) * ZCH
        pltpu.sync_copy(zbuf, acc.at[pl.ds(b, ZCH)])
        return carry

    lax.fori_loop(0, nz, zchunk, 0)

    if with_count:
        cnt16 = jnp.where(lax.iota(jnp.int32, 16) == 0,
                          jnp.full((16,), 1.0, _f32), jnp.zeros((16,), _f32))

        def initm(i, carry):
            msg_v[i, pl.ds(H, 16)] = cnt16
            return carry

        lax.fori_loop(0, CHUNK, initm, 0)

    plsc.subcore_barrier()

    nch = CH_BASE + jnp.where(wid < CH_REM, 1, 0)

    def chunk_body(k, carry):
        base = (wid + k * NW) * CHUNK
        pltpu.sync_copy(src_hbm.at[pl.ds(base, CHUNK)], src_idx)
        pltpu.sync_copy(dst_hbm.at[pl.ds(base, CHUNK)], dst_idx)
        pltpu.sync_copy(ea_hbm.at[pl.ds(2 * base, 2 * CHUNK)], ea_v)
        pltpu.async_copy(table_hbm.at[src_idx], rows_v, gsem).wait()

        def cbody(g, icarry):
            w16 = ea_v[pl.ds(g * 16, 16)]
            for j in range(8):
                i = g * 8 + j
                e0 = w16.at[jnp.full((16,), 2 * j, jnp.int32)].get(
                    mode="promise_in_bounds")
                e1 = w16.at[jnp.full((16,), 2 * j + 1, jnp.int32)].get(
                    mode="promise_in_bounds")
                for hh in range(H // 16):
                    r0 = rows_v[i, pl.ds(hh * 16, 16)]
                    r1 = rows_v[i, pl.ds(H + hh * 16, 16)]
                    r2 = rows_v[i, pl.ds(2 * H + hh * 16, 16)]
                    msg_v[i, pl.ds(hh * 16, 16)] = e0 * r0 + e1 * r1 + r2
            return icarry

        lax.fori_loop(0, CHUNK // 8, cbody, 0)
        pltpu.sync_copy(msg_v, acc.at[dst_idx], add=True)
        return carry

    lax.fori_loop(0, nch, chunk_body, 0)

    plsc.subcore_barrier()

    def dchunk(k, carry):
        b = (s + k * NS) * ZCH
        pltpu.sync_copy(acc.at[pl.ds(b, ZCH)], out_hbm.at[c, pl.ds(b, ZCH)])
        return carry

    lax.fori_loop(0, nz, dchunk, 0)


def _make_edge_call(W, with_count):
    mesh = plsc.VectorSubcoreMesh(core_axis_name="c", subcore_axis_name="s",
                                  num_cores=NC, num_subcores=NS)
    return pl.kernel(
        functools.partial(_edge_body, W, with_count),
        out_type=jax.ShapeDtypeStruct((NC, N, W), _f32),
        mesh=mesh,
        compiler_params=pltpu.CompilerParams(use_tc_tiling_on_sc=False),
        scratch_types=[
            pltpu.VMEM((CHUNK,), jnp.int32),
            pltpu.VMEM((CHUNK,), jnp.int32),
            pltpu.VMEM((2 * CHUNK,), _f32),
            pltpu.VMEM((CHUNK, TW), _f32),
            pltpu.VMEM((CHUNK, W), _f32),
            pltpu.VMEM((ZCH, W), _f32),
            pltpu.VMEM_SHARED((N, W), _f32),
            pltpu.SemaphoreType.DMA,
        ],
    )


_edge1 = _make_edge_call(H + 16, True)
_edge2 = _make_edge_call(H, False)


def _tc1_body(x_ref, w_ref, t_ref, r_ref):
    a = jnp.dot(x_ref[...], w_ref[...], preferred_element_type=_f32,
                precision=lax.Precision.HIGHEST)
    t_ref[...] = a[:, :TW]
    r_ref[...] = a[:, TW:]


_tc1 = pl.pallas_call(
    _tc1_body,
    out_shape=(jax.ShapeDtypeStruct((N, TW), _f32),
               jax.ShapeDtypeStruct((N, H), _f32)),
)


def _tc2_body(p_ref, rt_ref, b_ref, w_ref, t_ref, r_ref, inv_ref):
    p = p_ref[...]
    ssum = p[0] + p[1]
    inv = 1.0 / jnp.maximum(ssum[:, H:H + 1], 1.0)
    h = jnp.maximum(ssum[:, :H] * inv + rt_ref[...] + b_ref[...], 0.0)
    a2 = jnp.dot(h, w_ref[...], preferred_element_type=_f32,
                precision=lax.Precision.HIGHEST)
    t_ref[...] = a2[:, :TW]
    r_ref[...] = a2[:, TW:]
    inv_ref[...] = inv


_tc2 = pl.pallas_call(
    _tc2_body,
    out_shape=(jax.ShapeDtypeStruct((N, TW), _f32),
               jax.ShapeDtypeStruct((N, H), _f32),
               jax.ShapeDtypeStruct((N, 1), _f32)),
)


def _tc3_body(p_ref, rt_ref, b_ref, inv_ref, hcw_ref, hcb_ref, hrw_ref,
              hrb_ref, emb_ref, cls_ref, reg_ref):
    p = p_ref[...]
    ssum = p[0] + p[1]
    h2 = jnp.maximum(ssum * inv_ref[...] + rt_ref[...] + b_ref[...], 0.0)
    m = jnp.mean(h2, axis=0, keepdims=True)
    emb_ref[...] = m
    cls_ref[...] = jnp.dot(m, hcw_ref[...], preferred_element_type=_f32,
                precision=lax.Precision.HIGHEST) + hcb_ref[...]
    reg_ref[...] = jnp.dot(m, hrw_ref[...], preferred_element_type=_f32,
                precision=lax.Precision.HIGHEST) + hrb_ref[...]


_tc3 = pl.pallas_call(
    _tc3_body,
    out_shape=(jax.ShapeDtypeStruct((1, H), _f32),
               jax.ShapeDtypeStruct((1, 1), _f32),
               jax.ShapeDtypeStruct((1, 1), _f32)),
)


def kernel(x, edge_index, edge_attr, ew1_w, ew1_b, root1, bias1,
           ew2_w, ew2_b, root2, bias2,
           head_cls_w, head_cls_b, head_reg_w, head_reg_b):
    src = edge_index[0]
    dst = edge_index[1]
    wc1 = jnp.concatenate([ew1_w[0].reshape(D, H), ew1_w[1].reshape(D, H),
                           ew1_b.reshape(D, H), root1], axis=1)
    wc2 = jnp.concatenate([ew2_w[0].reshape(H, H), ew2_w[1].reshape(H, H),
                           ew2_b.reshape(H, H), root2], axis=1)

    ea1d = edge_attr.reshape(-1)
    t1, r1 = _tc1(x, wc1)
    p1 = _edge1(t1, src, dst, ea1d)
    t2, r2, inv = _tc2(p1, r1, bias1.reshape(1, H), wc2)
    p2 = _edge2(t2, src, dst, ea1d)
    emb, cls, reg = _tc3(p2, r2, bias2.reshape(1, H), inv,
                         head_cls_w[:H], head_cls_b.reshape(1, 1),
                         head_reg_w[:H], head_reg_b.reshape(1, 1))
    emb_full = jnp.concatenate([emb, jnp.zeros((1, H), _f32)], axis=1)
    return (cls.reshape(()), reg.reshape(()), emb_full.reshape(-1))


# speedup vs baseline: 6.4987x; 1.2702x over previous
import functools

import jax
import jax.numpy as jnp
from jax import lax
from jax.experimental import pallas as pl
from jax.experimental.pallas import tpu as pltpu
from jax.experimental.pallas import tpu_sc as plsc

N = 10000
E = 160000
D = 25
H = 32
TW = 3 * H

NC = 2
NS = 16
NW = NC * NS
CHUNK = 128
NCHUNKS = E // CHUNK
CH_BASE = NCHUNKS // NW
CH_REM = NCHUNKS % NW
ZCH = 200
NZCH = N // ZCH
ZBASE = NZCH // NS
ZREM = NZCH % NS

_f32 = jnp.float32


def _edge_body(W, with_count, table_hbm, eidx_hbm, ea_hbm, out_hbm,
               eidx_a, eidx_b, ea_a, ea_b, rows_a, rows_b, msg_a, msg_b,
               zbuf, acc, sem_a, sem_b):
    c = lax.axis_index("c")
    s = lax.axis_index("s")
    wid = s * NC + c

    zero16 = jnp.zeros((16,), _f32)

    def zrow(i, carry):
        for j in range(W // 16):
            zbuf[i, pl.ds(j * 16, 16)] = zero16
        return carry

    lax.fori_loop(0, ZCH, zrow, 0)
    nz = ZBASE + jnp.where(s < ZREM, 1, 0)

    def zchunk(k, carry):
        b = (s + k * NS) * ZCH
        pltpu.sync_copy(zbuf, acc.at[pl.ds(b, ZCH)])
        return carry

    lax.fori_loop(0, nz, zchunk, 0)

    if with_count:
        cnt16 = jnp.where(lax.iota(jnp.int32, 16) == 0,
                          jnp.full((16,), 1.0, _f32), jnp.zeros((16,), _f32))

        def initm(i, carry):
            msg_a[i, pl.ds(H, 16)] = cnt16
            msg_b[i, pl.ds(H, 16)] = cnt16
            return carry

        lax.fori_loop(0, CHUNK, initm, 0)

    plsc.subcore_barrier()

    nch = CH_BASE + jnp.where(wid < CH_REM, 1, 0)

    def fetch(j, eidx_v, ea_v, rows_v, sem):
        base = (wid + j * NW) * CHUNK
        pltpu.sync_copy(eidx_hbm.at[:, pl.ds(base, CHUNK)], eidx_v)
        pltpu.async_copy(table_hbm.at[eidx_v.at[0]], rows_v, sem)
        pltpu.sync_copy(ea_hbm.at[pl.ds(2 * base, 2 * CHUNK)], ea_v)

    def consume(eidx_v, ea_v, rows_v, msg_v, sem):
        pltpu.make_async_copy(table_hbm.at[pl.ds(0, CHUNK)], rows_v, sem).wait()

        def cbody(g, icarry):
            w16 = ea_v[pl.ds(g * 16, 16)]
            for j in range(8):
                i = g * 8 + j
                e0 = w16.at[jnp.full((16,), 2 * j, jnp.int32)].get(
                    mode="promise_in_bounds")
                e1 = w16.at[jnp.full((16,), 2 * j + 1, jnp.int32)].get(
                    mode="promise_in_bounds")
                for hh in range(H // 16):
                    r0 = rows_v[i, pl.ds(hh * 16, 16)]
                    r1 = rows_v[i, pl.ds(H + hh * 16, 16)]
                    r2 = rows_v[i, pl.ds(2 * H + hh * 16, 16)]
                    msg_v[i, pl.ds(hh * 16, 16)] = e0 * r0 + e1 * r1 + r2
            return icarry

        lax.fori_loop(0, CHUNK // 8, cbody, 0)
        pltpu.sync_copy(msg_v, acc.at[eidx_v.at[1]], add=True)

    fetch(0, eidx_a, ea_a, rows_a, sem_a)

    def pair(k2, carry):
        ja = 2 * k2
        jb = ja + 1

        @pl.when(jb < nch)
        def _():
            fetch(jb, eidx_b, ea_b, rows_b, sem_b)

        consume(eidx_a, ea_a, rows_a, msg_a, sem_a)

        @pl.when(jb < nch)
        def _():
            @pl.when(jb + 1 < nch)
            def _():
                fetch(jb + 1, eidx_a, ea_a, rows_a, sem_a)

            consume(eidx_b, ea_b, rows_b, msg_b, sem_b)

        return carry

    lax.fori_loop(0, (CH_BASE + 2) // 2, pair, 0)

    plsc.subcore_barrier()

    def dchunk(k, carry):
        b = (s + k * NS) * ZCH
        pltpu.sync_copy(acc.at[pl.ds(b, ZCH)], out_hbm.at[c, pl.ds(b, ZCH)])
        return carry

    lax.fori_loop(0, nz, dchunk, 0)


def _make_edge_call(W, with_count):
    mesh = plsc.VectorSubcoreMesh(core_axis_name="c", subcore_axis_name="s",
                                  num_cores=NC, num_subcores=NS)
    return pl.kernel(
        functools.partial(_edge_body, W, with_count),
        out_type=jax.ShapeDtypeStruct((NC, N, W), _f32),
        mesh=mesh,
        compiler_params=pltpu.CompilerParams(use_tc_tiling_on_sc=False),
        scratch_types=[
            pltpu.VMEM((2, CHUNK), jnp.int32),
            pltpu.VMEM((2, CHUNK), jnp.int32),
            pltpu.VMEM((2 * CHUNK,), _f32),
            pltpu.VMEM((2 * CHUNK,), _f32),
            pltpu.VMEM((CHUNK, TW), _f32),
            pltpu.VMEM((CHUNK, TW), _f32),
            pltpu.VMEM((CHUNK, W), _f32),
            pltpu.VMEM((CHUNK, W), _f32),
            pltpu.VMEM((ZCH, W), _f32),
            pltpu.VMEM_SHARED((N, W), _f32),
            pltpu.SemaphoreType.DMA,
            pltpu.SemaphoreType.DMA,
        ],
    )


_edge1 = _make_edge_call(H + 16, True)
_edge2 = _make_edge_call(H, False)


def _tc1_body(x_ref, w_ref, t_ref, r_ref):
    a = jnp.dot(x_ref[...], w_ref[...], preferred_element_type=_f32,
                precision=lax.Precision.HIGHEST)
    t_ref[...] = a[:, :TW]
    r_ref[...] = a[:, TW:]


_tc1 = pl.pallas_call(
    _tc1_body,
    out_shape=(jax.ShapeDtypeStruct((N, TW), _f32),
               jax.ShapeDtypeStruct((N, H), _f32)),
)


def _tc2_body(p_ref, rt_ref, b_ref, w_ref, t_ref, r_ref, inv_ref):
    p = p_ref[...]
    ssum = p[0] + p[1]
    inv = 1.0 / jnp.maximum(ssum[:, H:H + 1], 1.0)
    h = jnp.maximum(ssum[:, :H] * inv + rt_ref[...] + b_ref[...], 0.0)
    a2 = jnp.dot(h, w_ref[...], preferred_element_type=_f32,
                precision=lax.Precision.HIGHEST)
    t_ref[...] = a2[:, :TW]
    r_ref[...] = a2[:, TW:]
    inv_ref[...] = inv


_tc2 = pl.pallas_call(
    _tc2_body,
    out_shape=(jax.ShapeDtypeStruct((N, TW), _f32),
               jax.ShapeDtypeStruct((N, H), _f32),
               jax.ShapeDtypeStruct((N, 1), _f32)),
)


def _tc3_body(p_ref, rt_ref, b_ref, inv_ref, hcw_ref, hcb_ref, hrw_ref,
              hrb_ref, emb_ref, cls_ref, reg_ref):
    p = p_ref[...]
    ssum = p[0] + p[1]
    h2 = jnp.maximum(ssum * inv_ref[...] + rt_ref[...] + b_ref[...], 0.0)
    m = jnp.mean(h2, axis=0, keepdims=True)
    emb_ref[...] = m
    cls_ref[...] = jnp.dot(m, hcw_ref[...], preferred_element_type=_f32,
                precision=lax.Precision.HIGHEST) + hcb_ref[...]
    reg_ref[...] = jnp.dot(m, hrw_ref[...], preferred_element_type=_f32,
                precision=lax.Precision.HIGHEST) + hrb_ref[...]


_tc3 = pl.pallas_call(
    _tc3_body,
    out_shape=(jax.ShapeDtypeStruct((1, H), _f32),
               jax.ShapeDtypeStruct((1, 1), _f32),
               jax.ShapeDtypeStruct((1, 1), _f32)),
)


def kernel(x, edge_index, edge_attr, ew1_w, ew1_b, root1, bias1,
           ew2_w, ew2_b, root2, bias2,
           head_cls_w, head_cls_b, head_reg_w, head_reg_b):
    wc1 = jnp.concatenate([ew1_w[0].reshape(D, H), ew1_w[1].reshape(D, H),
                           ew1_b.reshape(D, H), root1], axis=1)
    wc2 = jnp.concatenate([ew2_w[0].reshape(H, H), ew2_w[1].reshape(H, H),
                           ew2_b.reshape(H, H), root2], axis=1)

    ea1d = edge_attr.reshape(-1)
    t1, r1 = _tc1(x, wc1)
    p1 = _edge1(t1, edge_index, ea1d)
    t2, r2, inv = _tc2(p1, r1, bias1.reshape(1, H), wc2)
    p2 = _edge2(t2, edge_index, ea1d)
    emb, cls, reg = _tc3(p2, r2, bias2.reshape(1, H), inv,
                         head_cls_w[:H], head_cls_b.reshape(1, 1),
                         head_reg_w[:H], head_reg_b.reshape(1, 1))
    emb_full = jnp.concatenate([emb, jnp.zeros((1, H), _f32)], axis=1)
    return (cls.reshape(()), reg.reshape(()), emb_full.reshape(-1))


# speedup vs baseline: 8.5374x; 1.3137x over previous
import functools

import jax
import jax.numpy as jnp
from jax import lax
from jax.experimental import pallas as pl
from jax.experimental.pallas import tpu as pltpu
from jax.experimental.pallas import tpu_sc as plsc

N = 10000
E = 160000
D = 25
H = 32
TW = 3 * H

NC = 2
NS = 16
NW = NC * NS
CHUNK = 128
NCHUNKS = E // CHUNK
CH_BASE = NCHUNKS // NW
CH_REM = NCHUNKS % NW
ZCH = 200
NZCH = N // ZCH
ZBASE = NZCH // NS
ZREM = NZCH % NS

_f32 = jnp.float32


def _edge_body(W, with_count, table_hbm, eidx_hbm, ea_hbm, out_hbm,
               eidx_a, eidx_b, ea_a, ea_b, rows_a, rows_b, msg_a, msg_b,
               zbuf, acc, sem_a, sem_b):
    c = lax.axis_index("c")
    s = lax.axis_index("s")
    wid = s * NC + c

    zero16 = jnp.zeros((16,), _f32)

    def zrow(i, carry):
        for j in range(W // 16):
            zbuf[i, pl.ds(j * 16, 16)] = zero16
        return carry

    lax.fori_loop(0, ZCH, zrow, 0)
    nz = ZBASE + jnp.where(s < ZREM, 1, 0)

    def zchunk(k, carry):
        b = (s + k * NS) * ZCH
        pltpu.sync_copy(zbuf, acc.at[pl.ds(b, ZCH)])
        return carry

    lax.fori_loop(0, nz, zchunk, 0)

    if with_count:
        cnt16 = jnp.where(lax.iota(jnp.int32, 16) == 0,
                          jnp.full((16,), 1.0, _f32), jnp.zeros((16,), _f32))

        def initm(i, carry):
            msg_a[i, pl.ds(H, 16)] = cnt16
            msg_b[i, pl.ds(H, 16)] = cnt16
            return carry

        lax.fori_loop(0, CHUNK, initm, 0)

    plsc.subcore_barrier()

    nch = CH_BASE + jnp.where(wid < CH_REM, 1, 0)

    def fetch(j, eidx_v, ea_v, rows_v, sem):
        base = (wid + j * NW) * CHUNK
        pltpu.sync_copy(eidx_hbm.at[:, pl.ds(base, CHUNK)], eidx_v)
        pltpu.async_copy(table_hbm.at[eidx_v.at[0]], rows_v, sem)
        pltpu.sync_copy(ea_hbm.at[pl.ds(2 * base, 2 * CHUNK)], ea_v)

    def consume(eidx_v, ea_v, rows_v, msg_v, sem):
        pltpu.make_async_copy(table_hbm.at[pl.ds(0, CHUNK)], rows_v, sem).wait()

        @plsc.parallel_loop(0, CHUNK // 8, unroll=2)
        def cbody(g):
            w16 = ea_v[pl.ds(g * 16, 16)]
            for j in range(8):
                i = g * 8 + j
                e0 = w16.at[jnp.full((16,), 2 * j, jnp.int32)].get(
                    mode="promise_in_bounds")
                e1 = w16.at[jnp.full((16,), 2 * j + 1, jnp.int32)].get(
                    mode="promise_in_bounds")
                for hh in range(H // 16):
                    r0 = rows_v[i, pl.ds(hh * 16, 16)]
                    r1 = rows_v[i, pl.ds(H + hh * 16, 16)]
                    r2 = rows_v[i, pl.ds(2 * H + hh * 16, 16)]
                    msg_v[i, pl.ds(hh * 16, 16)] = e0 * r0 + e1 * r1 + r2
        pltpu.sync_copy(msg_v, acc.at[eidx_v.at[1]], add=True)

    fetch(0, eidx_a, ea_a, rows_a, sem_a)

    def pair(k2, carry):
        ja = 2 * k2
        jb = ja + 1

        @pl.when(jb < nch)
        def _():
            fetch(jb, eidx_b, ea_b, rows_b, sem_b)

        consume(eidx_a, ea_a, rows_a, msg_a, sem_a)

        @pl.when(jb < nch)
        def _():
            @pl.when(jb + 1 < nch)
            def _():
                fetch(jb + 1, eidx_a, ea_a, rows_a, sem_a)

            consume(eidx_b, ea_b, rows_b, msg_b, sem_b)

        return carry

    lax.fori_loop(0, (CH_BASE + 2) // 2, pair, 0)

    plsc.subcore_barrier()

    def dchunk(k, carry):
        b = (s + k * NS) * ZCH
        pltpu.sync_copy(acc.at[pl.ds(b, ZCH)], out_hbm.at[c, pl.ds(b, ZCH)])
        return carry

    lax.fori_loop(0, nz, dchunk, 0)


def _make_edge_call(W, with_count):
    mesh = plsc.VectorSubcoreMesh(core_axis_name="c", subcore_axis_name="s",
                                  num_cores=NC, num_subcores=NS)
    return pl.kernel(
        functools.partial(_edge_body, W, with_count),
        out_type=jax.ShapeDtypeStruct((NC, N, W), _f32),
        mesh=mesh,
        compiler_params=pltpu.CompilerParams(use_tc_tiling_on_sc=False),
        scratch_types=[
            pltpu.VMEM((2, CHUNK), jnp.int32),
            pltpu.VMEM((2, CHUNK), jnp.int32),
            pltpu.VMEM((2 * CHUNK,), _f32),
            pltpu.VMEM((2 * CHUNK,), _f32),
            pltpu.VMEM((CHUNK, TW), _f32),
            pltpu.VMEM((CHUNK, TW), _f32),
            pltpu.VMEM((CHUNK, W), _f32),
            pltpu.VMEM((CHUNK, W), _f32),
            pltpu.VMEM((ZCH, W), _f32),
            pltpu.VMEM_SHARED((N, W), _f32),
            pltpu.SemaphoreType.DMA,
            pltpu.SemaphoreType.DMA,
        ],
    )


_edge1 = _make_edge_call(H + 16, True)
_edge2 = _make_edge_call(H, False)


def _tc1_body(x_ref, w_ref, t_ref, r_ref):
    a = jnp.dot(x_ref[...], w_ref[...], preferred_element_type=_f32,
                precision=lax.Precision.HIGHEST)
    t_ref[...] = a[:, :TW]
    r_ref[...] = a[:, TW:]


_tc1 = pl.pallas_call(
    _tc1_body,
    out_shape=(jax.ShapeDtypeStruct((N, TW), _f32),
               jax.ShapeDtypeStruct((N, H), _f32)),
)


def _tc2_body(p_ref, rt_ref, b_ref, w_ref, t_ref, r_ref, inv_ref):
    p = p_ref[...]
    ssum = p[0] + p[1]
    inv = 1.0 / jnp.maximum(ssum[:, H:H + 1], 1.0)
    h = jnp.maximum(ssum[:, :H] * inv + rt_ref[...] + b_ref[...], 0.0)
    a2 = jnp.dot(h, w_ref[...], preferred_element_type=_f32,
                precision=lax.Precision.HIGHEST)
    t_ref[...] = a2[:, :TW]
    r_ref[...] = a2[:, TW:]
    inv_ref[...] = inv


_tc2 = pl.pallas_call(
    _tc2_body,
    out_shape=(jax.ShapeDtypeStruct((N, TW), _f32),
               jax.ShapeDtypeStruct((N, H), _f32),
               jax.ShapeDtypeStruct((N, 1), _f32)),
)


def _tc3_body(p_ref, rt_ref, b_ref, inv_ref, hcw_ref, hcb_ref, hrw_ref,
              hrb_ref, emb_ref, cls_ref, reg_ref):
    p = p_ref[...]
    ssum = p[0] + p[1]
    h2 = jnp.maximum(ssum * inv_ref[...] + rt_ref[...] + b_ref[...], 0.0)
    m = jnp.mean(h2, axis=0, keepdims=True)
    emb_ref[...] = m
    cls_ref[...] = jnp.dot(m, hcw_ref[...], preferred_element_type=_f32,
                precision=lax.Precision.HIGHEST) + hcb_ref[...]
    reg_ref[...] = jnp.dot(m, hrw_ref[...], preferred_element_type=_f32,
                precision=lax.Precision.HIGHEST) + hrb_ref[...]


_tc3 = pl.pallas_call(
    _tc3_body,
    out_shape=(jax.ShapeDtypeStruct((1, H), _f32),
               jax.ShapeDtypeStruct((1, 1), _f32),
               jax.ShapeDtypeStruct((1, 1), _f32)),
)


def kernel(x, edge_index, edge_attr, ew1_w, ew1_b, root1, bias1,
           ew2_w, ew2_b, root2, bias2,
           head_cls_w, head_cls_b, head_reg_w, head_reg_b):
    wc1 = jnp.concatenate([ew1_w[0].reshape(D, H), ew1_w[1].reshape(D, H),
                           ew1_b.reshape(D, H), root1], axis=1)
    wc2 = jnp.concatenate([ew2_w[0].reshape(H, H), ew2_w[1].reshape(H, H),
                           ew2_b.reshape(H, H), root2], axis=1)

    ea1d = edge_attr.reshape(-1)
    t1, r1 = _tc1(x, wc1)
    p1 = _edge1(t1, edge_index, ea1d)
    t2, r2, inv = _tc2(p1, r1, bias1.reshape(1, H), wc2)
    p2 = _edge2(t2, edge_index, ea1d)
    emb, cls, reg = _tc3(p2, r2, bias2.reshape(1, H), inv,
                         head_cls_w[:H], head_cls_b.reshape(1, 1),
                         head_reg_w[:H], head_reg_b.reshape(1, 1))
    emb_full = jnp.concatenate([emb, jnp.zeros((1, H), _f32)], axis=1)
    return (cls.reshape(()), reg.reshape(()), emb_full.reshape(-1))


# speedup vs baseline: 8.5970x; 1.0070x over previous
import functools

import jax
import jax.numpy as jnp
from jax import lax
from jax.experimental import pallas as pl
from jax.experimental.pallas import tpu as pltpu
from jax.experimental.pallas import tpu_sc as plsc

N = 10000
E = 160000
D = 25
H = 32
TW = 3 * H

NC = 2
NS = 16
NW = NC * NS
CHUNK = 128
NCHUNKS = E // CHUNK
CH_BASE = NCHUNKS // NW
CH_REM = NCHUNKS % NW
ZCH = 200
NZCH = N // ZCH
ZBASE = NZCH // NS
ZREM = NZCH % NS

_f32 = jnp.float32


def _edge_body(W, with_count, table_hbm, eidx_hbm, ea_hbm, out_hbm,
               eidx_a, eidx_b, ea_a, ea_b, rows_a, rows_b, msg_a, msg_b,
               zbuf, acc, sem_a, sem_b):
    c = lax.axis_index("c")
    s = lax.axis_index("s")
    wid = s * NC + c

    zero16 = jnp.zeros((16,), _f32)

    def zrow(i, carry):
        for j in range(W // 16):
            zbuf[i, pl.ds(j * 16, 16)] = zero16
        return carry

    lax.fori_loop(0, ZCH, zrow, 0)
    nz = ZBASE + jnp.where(s < ZREM, 1, 0)

    def zchunk(k, carry):
        b = (s + k * NS) * ZCH
        pltpu.sync_copy(zbuf, acc.at[pl.ds(b, ZCH)])
        return carry

    lax.fori_loop(0, nz, zchunk, 0)

    if with_count:
        cnt16 = jnp.where(lax.iota(jnp.int32, 16) == 0,
                          jnp.full((16,), 1.0, _f32), jnp.zeros((16,), _f32))

        def initm(i, carry):
            msg_a[i, pl.ds(H, 16)] = cnt16
            msg_b[i, pl.ds(H, 16)] = cnt16
            return carry

        lax.fori_loop(0, CHUNK, initm, 0)

    plsc.subcore_barrier()

    nch = CH_BASE + jnp.where(wid < CH_REM, 1, 0)

    def fetch(j, eidx_v, ea_v, rows_v, sem):
        base = (wid + j * NW) * CHUNK
        pltpu.sync_copy(eidx_hbm.at[:, pl.ds(base, CHUNK)], eidx_v)
        pltpu.async_copy(table_hbm.at[eidx_v.at[0]], rows_v, sem)
        pltpu.sync_copy(ea_hbm.at[pl.ds(2 * base, 2 * CHUNK)], ea_v)

    def consume(eidx_v, ea_v, rows_v, msg_v, sem):
        pltpu.make_async_copy(table_hbm.at[pl.ds(0, CHUNK)], rows_v, sem).wait()

        @plsc.parallel_loop(0, CHUNK // 8, unroll=4)
        def cbody(g):
            w16 = ea_v[pl.ds(g * 16, 16)]
            for j in range(8):
                i = g * 8 + j
                e0 = w16.at[jnp.full((16,), 2 * j, jnp.int32)].get(
                    mode="promise_in_bounds")
                e1 = w16.at[jnp.full((16,), 2 * j + 1, jnp.int32)].get(
                    mode="promise_in_bounds")
                for hh in range(H // 16):
                    r0 = rows_v[i, pl.ds(hh * 16, 16)]
                    r1 = rows_v[i, pl.ds(H + hh * 16, 16)]
                    r2 = rows_v[i, pl.ds(2 * H + hh * 16, 16)]
                    msg_v[i, pl.ds(hh * 16, 16)] = e0 * r0 + e1 * r1 + r2
        pltpu.sync_copy(msg_v, acc.at[eidx_v.at[1]], add=True)

    fetch(0, eidx_a, ea_a, rows_a, sem_a)

    def pair(k2, carry):
        ja = 2 * k2
        jb = ja + 1

        @pl.when(jb < nch)
        def _():
            fetch(jb, eidx_b, ea_b, rows_b, sem_b)

        consume(eidx_a, ea_a, rows_a, msg_a, sem_a)

        @pl.when(jb < nch)
        def _():
            @pl.when(jb + 1 < nch)
            def _():
                fetch(jb + 1, eidx_a, ea_a, rows_a, sem_a)

            consume(eidx_b, ea_b, rows_b, msg_b, sem_b)

        return carry

    lax.fori_loop(0, (CH_BASE + 2) // 2, pair, 0)

    plsc.subcore_barrier()

    def dchunk(k, carry):
        b = (s + k * NS) * ZCH
        pltpu.sync_copy(acc.at[pl.ds(b, ZCH)], out_hbm.at[c, pl.ds(b, ZCH)])
        return carry

    lax.fori_loop(0, nz, dchunk, 0)


def _make_edge_call(W, with_count):
    mesh = plsc.VectorSubcoreMesh(core_axis_name="c", subcore_axis_name="s",
                                  num_cores=NC, num_subcores=NS)
    return pl.kernel(
        functools.partial(_edge_body, W, with_count),
        out_type=jax.ShapeDtypeStruct((NC, N, W), _f32),
        mesh=mesh,
        compiler_params=pltpu.CompilerParams(use_tc_tiling_on_sc=False),
        scratch_types=[
            pltpu.VMEM((2, CHUNK), jnp.int32),
            pltpu.VMEM((2, CHUNK), jnp.int32),
            pltpu.VMEM((2 * CHUNK,), _f32),
            pltpu.VMEM((2 * CHUNK,), _f32),
            pltpu.VMEM((CHUNK, TW), _f32),
            pltpu.VMEM((CHUNK, TW), _f32),
            pltpu.VMEM((CHUNK, W), _f32),
            pltpu.VMEM((CHUNK, W), _f32),
            pltpu.VMEM((ZCH, W), _f32),
            pltpu.VMEM_SHARED((N, W), _f32),
            pltpu.SemaphoreType.DMA,
            pltpu.SemaphoreType.DMA,
        ],
    )


_edge1 = _make_edge_call(H + 16, True)
_edge2 = _make_edge_call(H, False)


def _tc1_body(x_ref, w_ref, t_ref, r_ref):
    a = jnp.dot(x_ref[...], w_ref[...], preferred_element_type=_f32,
                precision=lax.Precision.HIGHEST)
    t_ref[...] = a[:, :TW]
    r_ref[...] = a[:, TW:]


_tc1 = pl.pallas_call(
    _tc1_body,
    out_shape=(jax.ShapeDtypeStruct((N, TW), _f32),
               jax.ShapeDtypeStruct((N, H), _f32)),
)


def _tc2_body(p_ref, rt_ref, b_ref, w_ref, t_ref, r_ref, inv_ref):
    p = p_ref[...]
    ssum = p[0] + p[1]
    inv = 1.0 / jnp.maximum(ssum[:, H:H + 1], 1.0)
    h = jnp.maximum(ssum[:, :H] * inv + rt_ref[...] + b_ref[...], 0.0)
    a2 = jnp.dot(h, w_ref[...], preferred_element_type=_f32,
                precision=lax.Precision.HIGHEST)
    t_ref[...] = a2[:, :TW]
    r_ref[...] = a2[:, TW:]
    inv_ref[...] = inv


_tc2 = pl.pallas_call(
    _tc2_body,
    out_shape=(jax.ShapeDtypeStruct((N, TW), _f32),
               jax.ShapeDtypeStruct((N, H), _f32),
               jax.ShapeDtypeStruct((N, 1), _f32)),
)


def _tc3_body(p_ref, rt_ref, b_ref, inv_ref, hcw_ref, hcb_ref, hrw_ref,
              hrb_ref, emb_ref, cls_ref, reg_ref):
    p = p_ref[...]
    ssum = p[0] + p[1]
    h2 = jnp.maximum(ssum * inv_ref[...] + rt_ref[...] + b_ref[...], 0.0)
    m = jnp.mean(h2, axis=0, keepdims=True)
    emb_ref[...] = m
    cls_ref[...] = jnp.dot(m, hcw_ref[...], preferred_element_type=_f32,
                precision=lax.Precision.HIGHEST) + hcb_ref[...]
    reg_ref[...] = jnp.dot(m, hrw_ref[...], preferred_element_type=_f32,
                precision=lax.Precision.HIGHEST) + hrb_ref[...]


_tc3 = pl.pallas_call(
    _tc3_body,
    out_shape=(jax.ShapeDtypeStruct((1, H), _f32),
               jax.ShapeDtypeStruct((1, 1), _f32),
               jax.ShapeDtypeStruct((1, 1), _f32)),
)


def kernel(x, edge_index, edge_attr, ew1_w, ew1_b, root1, bias1,
           ew2_w, ew2_b, root2, bias2,
           head_cls_w, head_cls_b, head_reg_w, head_reg_b):
    wc1 = jnp.concatenate([ew1_w[0].reshape(D, H), ew1_w[1].reshape(D, H),
                           ew1_b.reshape(D, H), root1], axis=1)
    wc2 = jnp.concatenate([ew2_w[0].reshape(H, H), ew2_w[1].reshape(H, H),
                           ew2_b.reshape(H, H), root2], axis=1)

    ea1d = edge_attr.reshape(-1)
    t1, r1 = _tc1(x, wc1)
    p1 = _edge1(t1, edge_index, ea1d)
    t2, r2, inv = _tc2(p1, r1, bias1.reshape(1, H), wc2)
    p2 = _edge2(t2, edge_index, ea1d)
    emb, cls, reg = _tc3(p2, r2, bias2.reshape(1, H), inv,
                         head_cls_w[:H], head_cls_b.reshape(1, 1),
                         head_reg_w[:H], head_reg_b.reshape(1, 1))
    emb_full = jnp.concatenate([emb, jnp.zeros((1, H), _f32)], axis=1)
    return (cls.reshape(()), reg.reshape(()), emb_full.reshape(-1))


# speedup vs baseline: 10.2075x; 1.1873x over previous
import functools

import jax
import jax.numpy as jnp
from jax import lax
from jax.experimental import pallas as pl
from jax.experimental.pallas import tpu as pltpu
from jax.experimental.pallas import tpu_sc as plsc

N = 10000
E = 160000
D = 25
H = 32
TW = 3 * H

NC = 2
NS = 16
NW = NC * NS
CHUNK = 128
NCHUNKS = E // CHUNK
CH_BASE = NCHUNKS // NW
CH_REM = NCHUNKS % NW
ZCH = 200
NZCH = N // ZCH
ZBASE = NZCH // NS
ZREM = NZCH % NS

_f32 = jnp.float32


def _edge_body(W, with_count, table_hbm, eidx_hbm, ea_hbm, out_hbm,
               eidx_a, eidx_b, ea_a, ea_b, rows_a, rows_b, msg_a, msg_b,
               zbuf, acc, sem_a, sem_b):
    c = lax.axis_index("c")
    s = lax.axis_index("s")
    wid = s * NC + c

    zero16 = jnp.zeros((16,), _f32)

    def zrow(i, carry):
        for j in range(W // 16):
            zbuf[i, pl.ds(j * 16, 16)] = zero16
        return carry

    lax.fori_loop(0, ZCH, zrow, 0)
    nz = ZBASE + jnp.where(s < ZREM, 1, 0)

    def zchunk(k, carry):
        b = (s + k * NS) * ZCH
        pltpu.sync_copy(zbuf, acc.at[pl.ds(b, ZCH)])
        return carry

    lax.fori_loop(0, nz, zchunk, 0)

    if with_count:
        cnt16 = jnp.where(lax.iota(jnp.int32, 16) == 0,
                          jnp.full((16,), 1.0, _f32), jnp.zeros((16,), _f32))

        def initm(i, carry):
            msg_a[i, pl.ds(H, 16)] = cnt16
            msg_b[i, pl.ds(H, 16)] = cnt16
            return carry

        lax.fori_loop(0, CHUNK, initm, 0)

    plsc.subcore_barrier()

    nch = CH_BASE + jnp.where(wid < CH_REM, 1, 0)

    def fetch(j, eidx_v, ea_v, rows_v, sem):
        base = (wid + j * NW) * CHUNK
        pltpu.sync_copy(eidx_hbm.at[:, pl.ds(base, CHUNK)], eidx_v)
        pltpu.async_copy(table_hbm.at[eidx_v.at[0]], rows_v, sem)
        pltpu.sync_copy(ea_hbm.at[:, pl.ds(base, CHUNK)], ea_v)

    def consume(eidx_v, ea_v, rows_v, msg_v, sem):
        pltpu.make_async_copy(table_hbm.at[pl.ds(0, CHUNK)], rows_v, sem).wait()

        @plsc.parallel_loop(0, CHUNK // 16, unroll=2)
        def cbody(g):
            a0 = ea_v[0, pl.ds(g * 16, 16)]
            a1 = ea_v[1, pl.ds(g * 16, 16)]
            for j in range(16):
                i = g * 16 + j
                cj = jnp.full((16,), j, jnp.int32)
                e0 = a0.at[cj].get(mode="promise_in_bounds")
                e1 = a1.at[cj].get(mode="promise_in_bounds")
                for hh in range(H // 16):
                    r0 = rows_v[i, pl.ds(hh * 16, 16)]
                    r1 = rows_v[i, pl.ds(H + hh * 16, 16)]
                    r2 = rows_v[i, pl.ds(2 * H + hh * 16, 16)]
                    msg_v[i, pl.ds(hh * 16, 16)] = e0 * r0 + e1 * r1 + r2
        pltpu.sync_copy(msg_v, acc.at[eidx_v.at[1]], add=True)

    fetch(0, eidx_a, ea_a, rows_a, sem_a)

    def pair(k2, carry):
        ja = 2 * k2
        jb = ja + 1

        @pl.when(jb < nch)
        def _():
            fetch(jb, eidx_b, ea_b, rows_b, sem_b)

        consume(eidx_a, ea_a, rows_a, msg_a, sem_a)

        @pl.when(jb < nch)
        def _():
            @pl.when(jb + 1 < nch)
            def _():
                fetch(jb + 1, eidx_a, ea_a, rows_a, sem_a)

            consume(eidx_b, ea_b, rows_b, msg_b, sem_b)

        return carry

    lax.fori_loop(0, (CH_BASE + 2) // 2, pair, 0)

    plsc.subcore_barrier()

    def dchunk(k, carry):
        b = (s + k * NS) * ZCH
        pltpu.sync_copy(acc.at[pl.ds(b, ZCH)], out_hbm.at[c, pl.ds(b, ZCH)])
        return carry

    lax.fori_loop(0, nz, dchunk, 0)


def _make_edge_call(W, with_count):
    mesh = plsc.VectorSubcoreMesh(core_axis_name="c", subcore_axis_name="s",
                                  num_cores=NC, num_subcores=NS)
    return pl.kernel(
        functools.partial(_edge_body, W, with_count),
        out_type=jax.ShapeDtypeStruct((NC, N, W), _f32),
        mesh=mesh,
        compiler_params=pltpu.CompilerParams(use_tc_tiling_on_sc=False),
        scratch_types=[
            pltpu.VMEM((2, CHUNK), jnp.int32),
            pltpu.VMEM((2, CHUNK), jnp.int32),
            pltpu.VMEM((2, CHUNK), _f32),
            pltpu.VMEM((2, CHUNK), _f32),
            pltpu.VMEM((CHUNK, TW), _f32),
            pltpu.VMEM((CHUNK, TW), _f32),
            pltpu.VMEM((CHUNK, W), _f32),
            pltpu.VMEM((CHUNK, W), _f32),
            pltpu.VMEM((ZCH, W), _f32),
            pltpu.VMEM_SHARED((N, W), _f32),
            pltpu.SemaphoreType.DMA,
            pltpu.SemaphoreType.DMA,
        ],
    )


_edge1 = _make_edge_call(H + 16, True)
_edge2 = _make_edge_call(H, False)


def _tc1_body(x_ref, w_ref, t_ref, r_ref):
    a = jnp.dot(x_ref[...], w_ref[...], preferred_element_type=_f32,
                precision=lax.Precision.HIGHEST)
    t_ref[...] = a[:, :TW]
    r_ref[...] = a[:, TW:]


_tc1 = pl.pallas_call(
    _tc1_body,
    out_shape=(jax.ShapeDtypeStruct((N, TW), _f32),
               jax.ShapeDtypeStruct((N, H), _f32)),
)


def _tc2_body(p_ref, rt_ref, b_ref, w_ref, t_ref, r_ref, inv_ref):
    p = p_ref[...]
    ssum = p[0] + p[1]
    inv = 1.0 / jnp.maximum(ssum[:, H:H + 1], 1.0)
    h = jnp.maximum(ssum[:, :H] * inv + rt_ref[...] + b_ref[...], 0.0)
    a2 = jnp.dot(h, w_ref[...], preferred_element_type=_f32,
                precision=lax.Precision.HIGHEST)
    t_ref[...] = a2[:, :TW]
    r_ref[...] = a2[:, TW:]
    inv_ref[...] = inv


_tc2 = pl.pallas_call(
    _tc2_body,
    out_shape=(jax.ShapeDtypeStruct((N, TW), _f32),
               jax.ShapeDtypeStruct((N, H), _f32),
               jax.ShapeDtypeStruct((N, 1), _f32)),
)


def _tc3_body(p_ref, rt_ref, b_ref, inv_ref, hcw_ref, hcb_ref, hrw_ref,
              hrb_ref, emb_ref, cls_ref, reg_ref):
    p = p_ref[...]
    ssum = p[0] + p[1]
    h2 = jnp.maximum(ssum * inv_ref[...] + rt_ref[...] + b_ref[...], 0.0)
    m = jnp.mean(h2, axis=0, keepdims=True)
    emb_ref[...] = m
    cls_ref[...] = jnp.dot(m, hcw_ref[...], preferred_element_type=_f32,
                precision=lax.Precision.HIGHEST) + hcb_ref[...]
    reg_ref[...] = jnp.dot(m, hrw_ref[...], preferred_element_type=_f32,
                precision=lax.Precision.HIGHEST) + hrb_ref[...]


_tc3 = pl.pallas_call(
    _tc3_body,
    out_shape=(jax.ShapeDtypeStruct((1, H), _f32),
               jax.ShapeDtypeStruct((1, 1), _f32),
               jax.ShapeDtypeStruct((1, 1), _f32)),
)


def kernel(x, edge_index, edge_attr, ew1_w, ew1_b, root1, bias1,
           ew2_w, ew2_b, root2, bias2,
           head_cls_w, head_cls_b, head_reg_w, head_reg_b):
    wc1 = jnp.concatenate([ew1_w[0].reshape(D, H), ew1_w[1].reshape(D, H),
                           ew1_b.reshape(D, H), root1], axis=1)
    wc2 = jnp.concatenate([ew2_w[0].reshape(H, H), ew2_w[1].reshape(H, H),
                           ew2_b.reshape(H, H), root2], axis=1)

    eat = edge_attr.T
    t1, r1 = _tc1(x, wc1)
    p1 = _edge1(t1, edge_index, eat)
    t2, r2, inv = _tc2(p1, r1, bias1.reshape(1, H), wc2)
    p2 = _edge2(t2, edge_index, eat)
    emb, cls, reg = _tc3(p2, r2, bias2.reshape(1, H), inv,
                         head_cls_w[:H], head_cls_b.reshape(1, 1),
                         head_reg_w[:H], head_reg_b.reshape(1, 1))
    emb_full = jnp.concatenate([emb, jnp.zeros((1, H), _f32)], axis=1)
    return (cls.reshape(()), reg.reshape(()), emb_full.reshape(-1))


# speedup vs baseline: 11.6858x; 1.1448x over previous
import functools

import jax
import jax.numpy as jnp
from jax import lax
from jax.experimental import pallas as pl
from jax.experimental.pallas import tpu as pltpu
from jax.experimental.pallas import tpu_sc as plsc

N = 10000
E = 160000
D = 25
H = 32
TW = 3 * H

NC = 2
NS = 16
NW = NC * NS
CHUNK = 128
NCHUNKS = E // CHUNK
CH_BASE = NCHUNKS // NW
CH_REM = NCHUNKS % NW
ZCH = 200
NZCH = N // ZCH
ZBASE = NZCH // NS
ZREM = NZCH % NS

_f32 = jnp.float32


def _edge_body(W, with_count, table_hbm, pk_hbm, out_hbm,
               pk_a, pk_b, rows_a, rows_b, msg_a, msg_b,
               zbuf, acc, sem_a, sem_b):
    c = lax.axis_index("c")
    s = lax.axis_index("s")
    wid = s * NC + c

    zero16 = jnp.zeros((16,), _f32)

    def zrow(i, carry):
        for j in range(W // 16):
            zbuf[i, pl.ds(j * 16, 16)] = zero16
        return carry

    lax.fori_loop(0, ZCH, zrow, 0)
    nz = ZBASE + jnp.where(s < ZREM, 1, 0)

    def zchunk(k, carry):
        b = (s + k * NS) * ZCH
        pltpu.sync_copy(zbuf, acc.at[pl.ds(b, ZCH)])
        return carry

    lax.fori_loop(0, nz, zchunk, 0)

    if with_count:
        cnt16 = jnp.where(lax.iota(jnp.int32, 16) == 0,
                          jnp.full((16,), 1.0, _f32), jnp.zeros((16,), _f32))

        def initm(i, carry):
            msg_a[i, pl.ds(H, 16)] = cnt16
            msg_b[i, pl.ds(H, 16)] = cnt16
            return carry

        lax.fori_loop(0, CHUNK, initm, 0)

    plsc.subcore_barrier()

    nch = CH_BASE + jnp.where(wid < CH_REM, 1, 0)

    def fetch(j, pk_v, rows_v, sem):
        base = (wid + j * NW) * CHUNK
        pltpu.sync_copy(pk_hbm.at[:, pl.ds(base, CHUNK)], pk_v)
        pltpu.async_copy(table_hbm.at[pk_v.at[0]], rows_v, sem)

    def consume(pk_v, rows_v, msg_v, sem):
        pltpu.make_async_copy(table_hbm.at[pl.ds(0, CHUNK)], rows_v, sem).wait()

        @plsc.parallel_loop(0, CHUNK // 16, unroll=2)
        def cbody(g):
            a0 = plsc.bitcast(pk_v[2, pl.ds(g * 16, 16)], _f32)
            a1 = plsc.bitcast(pk_v[3, pl.ds(g * 16, 16)], _f32)
            for j in range(16):
                i = g * 16 + j
                cj = jnp.full((16,), j, jnp.int32)
                e0 = a0.at[cj].get(mode="promise_in_bounds")
                e1 = a1.at[cj].get(mode="promise_in_bounds")
                for hh in range(H // 16):
                    r0 = rows_v[i, pl.ds(hh * 16, 16)]
                    r1 = rows_v[i, pl.ds(H + hh * 16, 16)]
                    r2 = rows_v[i, pl.ds(2 * H + hh * 16, 16)]
                    msg_v[i, pl.ds(hh * 16, 16)] = e0 * r0 + e1 * r1 + r2
        pltpu.sync_copy(msg_v, acc.at[pk_v.at[1]], add=True)

    fetch(0, pk_a, rows_a, sem_a)

    def pair(k2, carry):
        ja = 2 * k2
        jb = ja + 1

        @pl.when(jb < nch)
        def _():
            fetch(jb, pk_b, rows_b, sem_b)

        consume(pk_a, rows_a, msg_a, sem_a)

        @pl.when(jb < nch)
        def _():
            @pl.when(jb + 1 < nch)
            def _():
                fetch(jb + 1, pk_a, rows_a, sem_a)

            consume(pk_b, rows_b, msg_b, sem_b)

        return carry

    lax.fori_loop(0, (CH_BASE + 2) // 2, pair, 0)

    plsc.subcore_barrier()

    def dchunk(k, carry):
        b = (s + k * NS) * ZCH
        pltpu.sync_copy(acc.at[pl.ds(b, ZCH)], out_hbm.at[c, pl.ds(b, ZCH)])
        return carry

    lax.fori_loop(0, nz, dchunk, 0)


def _make_edge_call(W, with_count):
    mesh = plsc.VectorSubcoreMesh(core_axis_name="c", subcore_axis_name="s",
                                  num_cores=NC, num_subcores=NS)
    return pl.kernel(
        functools.partial(_edge_body, W, with_count),
        out_type=jax.ShapeDtypeStruct((NC, N, W), _f32),
        mesh=mesh,
        compiler_params=pltpu.CompilerParams(use_tc_tiling_on_sc=False,
                                            needs_layout_passes=False),
        scratch_types=[
            pltpu.VMEM((4, CHUNK), jnp.int32),
            pltpu.VMEM((4, CHUNK), jnp.int32),
            pltpu.VMEM((CHUNK, TW), _f32),
            pltpu.VMEM((CHUNK, TW), _f32),
            pltpu.VMEM((CHUNK, W), _f32),
            pltpu.VMEM((CHUNK, W), _f32),
            pltpu.VMEM((ZCH, W), _f32),
            pltpu.VMEM_SHARED((N, W), _f32),
            pltpu.SemaphoreType.DMA,
            pltpu.SemaphoreType.DMA,
        ],
    )


_edge1 = _make_edge_call(H + 16, True)
_edge2 = _make_edge_call(H, False)


def _tc1_body(x_ref, w_ref, t_ref, r_ref):
    a = jnp.dot(x_ref[...], w_ref[...], preferred_element_type=_f32,
                precision=lax.Precision.HIGHEST)
    t_ref[...] = a[:, :TW]
    r_ref[...] = a[:, TW:]


_tc1 = pl.pallas_call(
    _tc1_body,
    out_shape=(jax.ShapeDtypeStruct((N, TW), _f32),
               jax.ShapeDtypeStruct((N, H), _f32)),
)


def _tc2_body(p_ref, rt_ref, b_ref, w_ref, t_ref, r_ref, inv_ref):
    p = p_ref[...]
    ssum = p[0] + p[1]
    inv = 1.0 / jnp.maximum(ssum[:, H:H + 1], 1.0)
    h = jnp.maximum(ssum[:, :H] * inv + rt_ref[...] + b_ref[...], 0.0)
    a2 = jnp.dot(h, w_ref[...], preferred_element_type=_f32,
                precision=lax.Precision.HIGHEST)
    t_ref[...] = a2[:, :TW]
    r_ref[...] = a2[:, TW:]
    inv_ref[...] = inv


_tc2 = pl.pallas_call(
    _tc2_body,
    out_shape=(jax.ShapeDtypeStruct((N, TW), _f32),
               jax.ShapeDtypeStruct((N, H), _f32),
               jax.ShapeDtypeStruct((N, 1), _f32)),
)


def _tc3_body(p_ref, rt_ref, b_ref, inv_ref, hcw_ref, hcb_ref, hrw_ref,
              hrb_ref, emb_ref, cls_ref, reg_ref):
    p = p_ref[...]
    ssum = p[0] + p[1]
    h2 = jnp.maximum(ssum * inv_ref[...] + rt_ref[...] + b_ref[...], 0.0)
    m = jnp.mean(h2, axis=0, keepdims=True)
    emb_ref[...] = m
    cls_ref[...] = jnp.dot(m, hcw_ref[...], preferred_element_type=_f32,
                precision=lax.Precision.HIGHEST) + hcb_ref[...]
    reg_ref[...] = jnp.dot(m, hrw_ref[...], preferred_element_type=_f32,
                precision=lax.Precision.HIGHEST) + hrb_ref[...]


_tc3 = pl.pallas_call(
    _tc3_body,
    out_shape=(jax.ShapeDtypeStruct((1, H), _f32),
               jax.ShapeDtypeStruct((1, 1), _f32),
               jax.ShapeDtypeStruct((1, 1), _f32)),
)


def kernel(x, edge_index, edge_attr, ew1_w, ew1_b, root1, bias1,
           ew2_w, ew2_b, root2, bias2,
           head_cls_w, head_cls_b, head_reg_w, head_reg_b):
    wc1 = jnp.concatenate([ew1_w[0].reshape(D, H), ew1_w[1].reshape(D, H),
                           ew1_b.reshape(D, H), root1], axis=1)
    wc2 = jnp.concatenate([ew2_w[0].reshape(H, H), ew2_w[1].reshape(H, H),
                           ew2_b.reshape(H, H), root2], axis=1)

    packed = jnp.concatenate(
        [edge_index, lax.bitcast_convert_type(edge_attr.T, jnp.int32)], axis=0)
    t1, r1 = _tc1(x, wc1)
    p1 = _edge1(t1, packed)
    t2, r2, inv = _tc2(p1, r1, bias1.reshape(1, H), wc2)
    p2 = _edge2(t2, packed)
    emb, cls, reg = _tc3(p2, r2, bias2.reshape(1, H), inv,
                         head_cls_w[:H], head_cls_b.reshape(1, 1),
                         head_reg_w[:H], head_reg_b.reshape(1, 1))
    emb_full = jnp.concatenate([emb, jnp.zeros((1, H), _f32)], axis=1)
    return (cls.reshape(()), reg.reshape(()), emb_full.reshape(-1))
